# baseline (speedup 1.0000x reference)
import functools

import jax
import jax.numpy as jnp
from jax.experimental import pallas as pl
from jax.experimental.pallas import tpu as pltpu

N = 10000
E = 160000
F = 256
H = 8

_NP = 10240


def _matmul_body(a_ref, w_ref, o_ref):
    o_ref[...] = jnp.dot(a_ref[...], w_ref[...],
                         preferred_element_type=jnp.float32)


def _matmul(a, w_t, block_m=256):
    m, k = a.shape
    p = w_t.shape[1]
    grid = (m // block_m,)
    return pl.pallas_call(
        _matmul_body,
        grid=grid,
        in_specs=[
            pl.BlockSpec((block_m, k), lambda i: (i, 0)),
            pl.BlockSpec((k, p), lambda i: (0, 0)),
        ],
        out_specs=pl.BlockSpec((block_m, p), lambda i: (i, 0)),
        out_shape=jax.ShapeDtypeStruct((m, p), jnp.float32),
    )(a, w_t)


def _gat_layer(xpad, edge_index, Wl, Wr, att, b):
    src = edge_index[0]
    dst = edge_index[1]
    Hh, C = att.shape
    xl = _matmul(xpad, Wl.T)[:N].reshape(N, Hh, C)
    xr = _matmul(xpad, Wr.T)[:N].reshape(N, Hh, C)
    e = jax.nn.leaky_relu(xl[src] + xr[dst], 0.2)
    logits = jnp.einsum('ehc,hc->eh', e, att)
    m = jax.ops.segment_max(logits, dst, num_segments=N)
    m = jnp.where(jnp.isfinite(m), m, 0.0)
    ex = jnp.exp(logits - m[dst])
    den = jax.ops.segment_sum(ex, dst, num_segments=N)
    alpha = ex / (den[dst] + 1e-16)
    msg = xl[src] * alpha[:, :, None]
    out = jax.ops.segment_sum(msg, dst, num_segments=N)
    return out.mean(axis=1) + b


def kernel(x, edge_index, Wl1, Wr1, att1, b1, Wl2, Wr2, att2, b2, Wlin, blin):
    xpad = jnp.pad(x, ((0, _NP - N), (0, 0)))
    residual1 = _matmul(xpad, Wlin.T)[:N] + blin
    out = _gat_layer(xpad, edge_index, Wl1, Wr1, att1, b1)
    out = jax.nn.relu(out)
    opad = jnp.pad(out, ((0, _NP - N), (0, 0)))
    out = _gat_layer(opad, edge_index, Wl2, Wr2, att2, b2)
    return out + residual1


# speedup vs baseline: 1.0001x; 1.0001x over previous
import jax
import jax.numpy as jnp
from jax.experimental import pallas as pl

N = 10000
NPAD = 10240


def _matmul_body(a_ref, w_ref, o_ref):
    o_ref[...] = jnp.dot(a_ref[...], w_ref[...],
                         preferred_element_type=jnp.float32)


def _matmul(a, w_t, block_m=256):
    m, k = a.shape
    p = w_t.shape[1]
    return pl.pallas_call(
        _matmul_body,
        grid=(m // block_m,),
        in_specs=[
            pl.BlockSpec((block_m, k), lambda i: (i, 0)),
            pl.BlockSpec((k, p), lambda i: (0, 0)),
        ],
        out_specs=pl.BlockSpec((block_m, p), lambda i: (i, 0)),
        out_shape=jax.ShapeDtypeStruct((m, p), jnp.float32),
    )(a, w_t)


def _edge_stage(xl, xr, src, dst, att, b):
    n = xl.shape[0]
    e = jax.nn.leaky_relu(xl[src] + xr[dst], 0.2)
    logits = jnp.einsum('ehc,hc->eh', e, att)
    m = jax.ops.segment_max(logits, dst, num_segments=n)
    m = jnp.where(jnp.isfinite(m), m, 0.0)
    ex = jnp.exp(logits - m[dst])
    den = jax.ops.segment_sum(ex, dst, num_segments=n)
    alpha = ex / (den[dst] + 1e-16)
    msg = xl[src] * alpha[:, :, None]
    out = jax.ops.segment_sum(msg, dst, num_segments=n)
    return out.mean(axis=1) + b


def kernel(x, edge_index, Wl1, Wr1, att1, b1, Wl2, Wr2, att2, b2, Wlin, blin):
    H = att1.shape[0]
    C1 = att1.shape[1]
    C2 = att2.shape[1]
    xpad = jnp.pad(x, ((0, NPAD - N), (0, 0)))
    src = edge_index[0]
    dst = edge_index[1]

    resid = _matmul(xpad, Wlin.T)[:N] + blin
    xl1 = _matmul(xpad, Wl1.T)[:N].reshape(N, H, C1)
    xr1 = _matmul(xpad, Wr1.T)[:N].reshape(N, H, C1)
    h1 = jax.nn.relu(_edge_stage(xl1, xr1, src, dst, att1, b1))
    h1p = jnp.pad(h1, ((0, NPAD - N), (0, 0)))
    xl2 = _matmul(h1p, Wl2.T)[:N].reshape(N, H, C2)
    xr2 = _matmul(h1p, Wr2.T)[:N].reshape(N, H, C2)
    out = _edge_stage(xl2, xr2, src, dst, att2, b2)
    return out + resid


# speedup vs baseline: 4.9711x; 4.9707x over previous
import jax
import jax.numpy as jnp
from jax import lax
from jax.experimental import pallas as pl
from jax.experimental.pallas import tpu as pltpu
from jax.experimental.pallas import tpu_sc as plsc

N = 10000
E = 160000
F = 256
H = 8

NW = 32
NPW = 320
NPAD = NW * NPW
RP_LEN = NPAD + 16
E_PAD = E + 32
NEG = -1e30


def _matmul_body(a_ref, w_ref, o_ref):
    o_ref[...] = jnp.dot(a_ref[...], w_ref[...],
                         preferred_element_type=jnp.float32)


def _matmul(a, w_t, block_m=256):
    m, k = a.shape
    p = w_t.shape[1]
    return pl.pallas_call(
        _matmul_body,
        grid=(m // block_m,),
        in_specs=[
            pl.BlockSpec((block_m, k), lambda i: (i, 0)),
            pl.BlockSpec((k, p), lambda i: (0, 0)),
        ],
        out_specs=pl.BlockSpec((block_m, p), lambda i: (i, 0)),
        out_shape=jax.ShapeDtypeStruct((m, p), jnp.float32),
    )(a, w_t)


def _make_edge_stage(C, with_resid):
    KC = C // 16
    mesh = plsc.VectorSubcoreMesh(core_axis_name="c", subcore_axis_name="s")

    def body(xl_hbm, xr_hbm, rp_hbm, ssrc_hbm, att_hbm, b_hbm, *rest):
        if with_resid:
            (resid_hbm, out_hbm, rows_v, acc_v, xr_row_v, att_v, b_v,
             resid_row_v, out_row_v, rp_v, idxwin_v, ptmp_v, w_v, sem) = rest
        else:
            (out_hbm, rows_v, acc_v, xr_row_v, att_v, b_v,
             out_row_v, rp_v, idxwin_v, ptmp_v, w_v, sem) = rest

        wid = lax.axis_index("s") * 2 + lax.axis_index("c")
        n0 = pl.multiple_of(wid * NPW, 8)
        pltpu.sync_copy(rp_hbm.at[pl.ds(n0, NPW + 16)], rp_v)
        pltpu.sync_copy(att_hbm, att_v)
        pltpu.sync_copy(b_hbm, b_v)

        zeros16 = jnp.zeros((16,), jnp.float32)
        negs16 = jnp.full((16,), NEG, jnp.float32)
        hiota = lax.iota(jnp.int32, 16)

        def zh(h, _):
            def zk(k, _):
                acc_v[h, pl.ds(k * 16, 16)] = zeros16
                return 0
            return lax.fori_loop(0, KC, zk, 0)
        lax.fori_loop(0, H, zh, 0)

        def node_body(i, _):
            n = n0 + i
            rp_pair = rp_v[pl.ds(i, 16)]
            start = rp_pair[0]
            end = rp_pair[1]
            deg = end - start
            pltpu.sync_copy(xr_hbm.at[n], xr_row_v)
            if with_resid:
                pltpu.sync_copy(resid_hbm.at[n], resid_row_v)

            def chunk_body(jc, carry):
                mv, sv = carry
                base = start + jc * 16
                cnt = end - base
                abase = pl.multiple_of((base // 8) * 8, 8)
                off = base - abase
                pltpu.sync_copy(ssrc_hbm.at[pl.ds(abase, 32)], idxwin_v)
                idx16 = plsc.load_gather(idxwin_v, [off + hiota])
                pltpu.async_copy(xl_hbm.at[idx16], rows_v, sem).wait()

                valid = hiota < cnt

                def hbody(h, hcarry):
                    cmaxv, wsumv = hcarry

                    def kbody(k, parts):
                        xrv = xr_row_v[h, pl.ds(k * 16, 16)]
                        attv = att_v[h, pl.ds(k * 16, 16)]
                        out = []
                        for r in range(16):
                            t = rows_v[r, h, pl.ds(k * 16, 16)] + xrv
                            t = jnp.maximum(t, 0.2 * t)
                            out.append(parts[r] + t * attv)
                        return tuple(out)
                    parts = lax.fori_loop(0, KC, kbody,
                                          tuple(zeros16 for _ in range(16)))
                    for r in range(16):
                        ptmp_v[r, :] = parts[r]

                    def cbody(c, lg):
                        return lg + plsc.load_gather(
                            ptmp_v, [hiota, jnp.full((16,), 0, jnp.int32) + c])
                    lgt = lax.fori_loop(0, 16, cbody, zeros16)
                    lgt = jnp.where(valid, lgt, negs16)
                    cmax = jnp.max(lgt)
                    oh = hiota == h
                    mh = jnp.where(oh, mv, negs16)
                    mh = jnp.max(mh)
                    new_m = jnp.maximum(mh, cmax)
                    wv = jnp.exp(lgt - new_m)
                    wv = jnp.where(valid, wv, zeros16)
                    w_v[h, :] = wv
                    cmaxv = jnp.where(oh, new_m, cmaxv)
                    wsumv = jnp.where(oh, jnp.sum(wv), wsumv)
                    return (cmaxv, wsumv)

                new_mv, wsumv = lax.fori_loop(0, H, hbody, (negs16, zeros16))
                new_mv = jnp.maximum(new_mv, mv)
                scalev = jnp.exp(mv - new_mv)
                sv = sv * scalev + wsumv

                def habody(h, _):
                    scs = jnp.where(hiota == h, scalev, zeros16)
                    sc = jnp.sum(scs)
                    wrow = w_v[h, :]
                    ws = [wrow[r] for r in range(16)]

                    def kabody(k, _):
                        accv = acc_v[h, pl.ds(k * 16, 16)] * sc
                        for r in range(16):
                            accv = accv + (rows_v[r, h, pl.ds(k * 16, 16)]
                                           * ws[r])
                        acc_v[h, pl.ds(k * 16, 16)] = accv
                        return 0
                    lax.fori_loop(0, KC, kabody, 0)
                    return 0
                lax.fori_loop(0, H, habody, 0)
                return (new_mv, sv)

            nchunks = (deg + 15) // 16
            mv, sv = lax.fori_loop(0, nchunks, chunk_body, (negs16, zeros16))

            dmask = (deg > 0).astype(jnp.float32)
            numv = zeros16 + dmask * jnp.float32(1.0 / H)
            inv = numv / (sv + 1e-16)

            def f0body(k, _):
                o = b_v[pl.ds(k * 16, 16)]
                if with_resid:
                    o = o + resid_row_v[pl.ds(k * 16, 16)]
                out_row_v[pl.ds(k * 16, 16)] = o
                return 0
            lax.fori_loop(0, KC, f0body, 0)

            def fhbody(h, _):
                ivs = jnp.where(hiota == h, inv, zeros16)
                iv = jnp.sum(ivs)

                def fkbody(k, _):
                    o = out_row_v[pl.ds(k * 16, 16)]
                    o = o + acc_v[h, pl.ds(k * 16, 16)] * iv
                    out_row_v[pl.ds(k * 16, 16)] = o
                    return 0
                lax.fori_loop(0, KC, fkbody, 0)
                return 0
            lax.fori_loop(0, H, fhbody, 0)

            if not with_resid:
                def frbody(k, _):
                    o = out_row_v[pl.ds(k * 16, 16)]
                    out_row_v[pl.ds(k * 16, 16)] = jnp.maximum(o, 0.0)
                    return 0
                lax.fori_loop(0, KC, frbody, 0)

            pltpu.sync_copy(out_row_v, out_hbm.at[n])
            return 0

        lax.fori_loop(0, NPW, node_body, 0)

    scratch = [
        pltpu.VMEM((16, H, C), jnp.float32),
        pltpu.VMEM((H, C), jnp.float32),
        pltpu.VMEM((H, C), jnp.float32),
        pltpu.VMEM((H, C), jnp.float32),
        pltpu.VMEM((C,), jnp.float32),
    ]
    if with_resid:
        scratch.append(pltpu.VMEM((C,), jnp.float32))
    scratch += [
        pltpu.VMEM((C,), jnp.float32),
        pltpu.VMEM((NPW + 16,), jnp.int32),
        pltpu.VMEM((32,), jnp.int32),
        pltpu.VMEM((16, 16), jnp.float32),
        pltpu.VMEM((H, 16), jnp.float32),
        pltpu.SemaphoreType.DMA,
    ]
    return pl.kernel(
        body,
        out_type=jax.ShapeDtypeStruct((NPAD, C), jnp.float32),
        mesh=mesh,
        scratch_types=scratch,
        compiler_params=pltpu.CompilerParams(needs_layout_passes=False),
    )


def kernel(x, edge_index, Wl1, Wr1, att1, b1, Wl2, Wr2, att2, b2, Wlin, blin):
    C1 = att1.shape[1]
    C2 = att2.shape[1]
    xpad = jnp.pad(x, ((0, NPAD - N), (0, 0)))

    src = edge_index[0]
    dst = edge_index[1]
    sdst, ssrc = lax.sort_key_val(dst, src)
    rp = jnp.searchsorted(sdst, jnp.arange(RP_LEN, dtype=jnp.int32),
                          side="left").astype(jnp.int32)
    ssrc = jnp.pad(ssrc, (0, E_PAD - E))

    resid = _matmul(xpad, Wlin.T)

    xl1 = _matmul(xpad, Wl1.T).reshape(NPAD, H, C1)
    xr1 = _matmul(xpad, Wr1.T).reshape(NPAD, H, C1)
    h1 = _make_edge_stage(C1, False)(xl1, xr1, rp, ssrc, att1, b1)

    xl2 = _matmul(h1, Wl2.T).reshape(NPAD, H, C2)
    xr2 = _matmul(h1, Wr2.T).reshape(NPAD, H, C2)
    out = _make_edge_stage(C2, True)(
        xl2, xr2, rp, ssrc, att2, b2 + blin, resid)
    return out[:N]
